# Initial kernel scaffold; baseline (speedup 1.0000x reference)
import jax
import jax.numpy as jnp
from jax.experimental import pallas as pl


def kernel(x, edge_index, W1_0, b1_0, alpha_0, gamma_0, beta_0, W2_0, b2_0, W1_1, b1_1, alpha_1, gamma_1, beta_1, W2_1, b2_1):
    raise NotImplementedError("write your pallas kernel here")


# speedup vs baseline: 5.1628x; 5.1628x over previous
import functools

import jax
import jax.numpy as jnp
from jax import lax
from jax.experimental import pallas as pl
from jax.experimental.pallas import tpu as pltpu
from jax.experimental.pallas import tpu_sc as plsc

_N = 10000
_E = 320000
_D = 128
_NPAD = 10240
_CH = 80
_NTILES = 32
_EPT = _E // _NTILES
_CHUNKS = _EPT // _CH
_RPT = _NPAD // 16


def _segment_sum_sc(h, src, dst):
    mesh = plsc.VectorSubcoreMesh(core_axis_name="c", subcore_axis_name="s")

    @functools.partial(
        pl.kernel,
        out_type=jax.ShapeDtypeStruct((2, _NPAD, _D), jnp.float32),
        mesh=mesh,
        scratch_types=[
            pltpu.VMEM((_CH,), jnp.int32),
            pltpu.VMEM((_CH,), jnp.int32),
            pltpu.VMEM((_CH, _D), jnp.float32),
            pltpu.VMEM((16, _D), jnp.float32),
            pltpu.VMEM_SHARED((_NPAD, _D), jnp.float32),
            pltpu.SemaphoreType.DMA,
        ],
    )
    def k(h_hbm, src_hbm, dst_hbm, out_hbm, sidx, didx, rows, zbuf, acc, sem):
        cid = lax.axis_index("c")
        sid = lax.axis_index("s")
        wid = cid * 16 + sid

        def zstore(i, carry):
            zbuf[i // 8, pl.ds((i % 8) * 16, 16)] = jnp.zeros((16,), jnp.float32)
            return carry
        lax.fori_loop(0, 16 * (_D // 16), zstore, 0)

        def zcopy(j, carry):
            pltpu.sync_copy(zbuf, acc.at[pl.ds(sid * _RPT + j * 16, 16)])
            return carry
        lax.fori_loop(0, _RPT // 16, zcopy, 0)
        plsc.subcore_barrier()

        def body(i, carry):
            base = wid * _EPT + i * _CH
            pltpu.sync_copy(src_hbm.at[pl.ds(base, _CH)], sidx)
            pltpu.sync_copy(dst_hbm.at[pl.ds(base, _CH)], didx)
            pltpu.async_copy(h_hbm.at[sidx], rows, sem).wait()
            pltpu.sync_copy(rows, acc.at[didx], add=True)
            return carry
        lax.fori_loop(0, _CHUNKS, body, 0)
        plsc.subcore_barrier()

        pltpu.sync_copy(acc.at[pl.ds(sid * _RPT, _RPT)],
                        out_hbm.at[cid, pl.ds(sid * _RPT, _RPT)])

    return k(h, src, dst)


def _dense_body(x_ref, agg_ref, w1_ref, b1_ref, al_ref, g_ref, be_ref,
                w2_ref, b2_ref, out_ref):
    h = x_ref[...] + agg_ref[0, :_N, :] + agg_ref[1, :_N, :]
    h = jnp.dot(h, w1_ref[...], preferred_element_type=jnp.float32) + b1_ref[...]
    mean = jnp.mean(h, axis=0, keepdims=True)
    cen = h - al_ref[...] * mean
    var = jnp.mean(cen * cen, axis=0, keepdims=True)
    h = g_ref[...] * cen / jnp.sqrt(var + 1e-5) + be_ref[...]
    h = jnp.maximum(h, 0.0)
    h = jnp.dot(h, w2_ref[...], preferred_element_type=jnp.float32) + b2_ref[...]
    out_ref[...] = jnp.maximum(h, 0.0)


def _dense_layer(x, agg, W1, b1, alpha, gamma, beta, W2, b2):
    return pl.pallas_call(
        _dense_body,
        out_shape=jax.ShapeDtypeStruct((_N, _D), jnp.float32),
    )(x, agg, W1, b1.reshape(1, _D), alpha.reshape(1, _D),
      gamma.reshape(1, _D), beta.reshape(1, _D), W2, b2.reshape(1, _D))


def kernel(x, edge_index, W1_0, b1_0, alpha_0, gamma_0, beta_0, W2_0, b2_0,
           W1_1, b1_1, alpha_1, gamma_1, beta_1, W2_1, b2_1):
    src = edge_index[0]
    dst = edge_index[1]
    agg0 = _segment_sum_sc(x, src, dst)
    h = _dense_layer(x, agg0, W1_0, b1_0, alpha_0, gamma_0, beta_0, W2_0, b2_0)
    agg1 = _segment_sum_sc(h, src, dst)
    h = _dense_layer(h, agg1, W1_1, b1_1, alpha_1, gamma_1, beta_1, W2_1, b2_1)
    return h
